# baseline (speedup 1.0000x reference)
import jax
import jax.numpy as jnp
from jax.experimental import pallas as pl
from jax.experimental.pallas import tpu as pltpu

_BS = 64
_N = 3000
_NPAD = 3072
_LC = 1024
_N_LC = _NPAD // _LC
_JC = 8
_SYM = (12, 15, 18, 19, 20)
_PADVAL = 1e15


def _loss_kernel(params_ref, mpT_ref, tgtT_ref, tgt_ref, out_ref, r2_ref):
    mpx = mpT_ref[0, 0:1, :]
    mpy = mpT_ref[0, 1:2, :]
    mpz = mpT_ref[0, 2:3, :]

    def p(k):
        return params_ref[0, 0, k]

    tfx = mpx * p(0) + mpy * p(3) + mpz * p(6) + p(9)
    tfy = mpx * p(1) + mpy * p(4) + mpz * p(7) + p(10)
    tfz = mpx * p(2) + mpy * p(5) + mpz * p(8) + p(11)

    lane = jax.lax.broadcasted_iota(jnp.int32, (1, _NPAD), 1)
    lvalid = (lane < _N).astype(jnp.float32)

    m = p(12)

    @pl.when(m > 0.5)
    def _sym():
        tg0 = tgt_ref[0, :, 0:1]
        tg1 = tgt_ref[0, :, 1:2]
        tg2 = tgt_ref[0, :, 2:3]
        r2_ref[:, :] = tg0 * tg0 + tg1 * tg1 + tg2 * tg2

        total = jnp.zeros((1, 1), dtype=jnp.float32)
        for c in range(_N_LC):
            sl = slice(c * _LC, (c + 1) * _LC)
            bx = jnp.broadcast_to(-2.0 * tfx[:, sl], (_JC, _LC))
            by = jnp.broadcast_to(-2.0 * tfy[:, sl], (_JC, _LC))
            bz = jnp.broadcast_to(-2.0 * tfz[:, sl], (_JC, _LC))

            def body(j, acc):
                r0 = j * _JC
                cx = tgt_ref[0, pl.ds(r0, _JC), 0:1]
                cy = tgt_ref[0, pl.ds(r0, _JC), 1:2]
                cz = tgt_ref[0, pl.ds(r0, _JC), 2:3]
                cr = r2_ref[pl.ds(r0, _JC), 0:1]
                v = cx * bx + cy * by
                v = v + cz * bz
                v = v + cr
                return jnp.minimum(acc, v)

            minacc = jax.lax.fori_loop(
                0, _NPAD // _JC, body,
                jnp.full((_JC, _LC), jnp.inf, dtype=jnp.float32),
                unroll=4)
            mrow = jnp.min(minacc, axis=0, keepdims=True)
            q2 = (tfx[:, sl] * tfx[:, sl] + tfy[:, sl] * tfy[:, sl]
                  + tfz[:, sl] * tfz[:, sl])
            d2 = jnp.maximum(mrow + q2, 0.0)
            total = total + jnp.sum(jnp.sqrt(d2) * lvalid[:, sl],
                                    axis=1, keepdims=True)
        out_ref[0] = total

    @pl.when(m <= 0.5)
    def _plain():
        dx = tfx - tgtT_ref[0, 0:1, :]
        dy = tfy - tgtT_ref[0, 1:2, :]
        dz = tfz - tgtT_ref[0, 2:3, :]
        d2 = dx * dx + dy * dy + dz * dz
        s = jnp.sum(jnp.sqrt(d2) * lvalid, axis=1, keepdims=True)
        out_ref[0] = s


def kernel(pred_r, pred_t, target, model_points, idx):
    bs, num_p, _ = target.shape

    q = pred_r / jnp.linalg.norm(pred_r, axis=1, keepdims=True)
    w, x, y, z = q[:, 0], q[:, 1], q[:, 2], q[:, 3]
    r00 = 1.0 - 2.0 * (y * y + z * z)
    r01 = 2.0 * (x * y - w * z)
    r02 = 2.0 * (x * z + w * y)
    r10 = 2.0 * (x * y + w * z)
    r11 = 1.0 - 2.0 * (x * x + z * z)
    r12 = 2.0 * (y * z - w * x)
    r20 = 2.0 * (x * z - w * y)
    r21 = 2.0 * (y * z + w * x)
    r22 = 1.0 - 2.0 * (x * x + y * y)
    sym = jnp.asarray(_SYM, dtype=idx.dtype)
    mask = (idx[:, 0][:, None] == sym[None, :]).any(axis=1).astype(jnp.float32)
    zeros = jnp.zeros_like(w)
    params = jnp.stack(
        [r00, r01, r02, r10, r11, r12, r20, r21, r22,
         pred_t[:, 0], pred_t[:, 1], pred_t[:, 2], mask, zeros, zeros, zeros],
        axis=1).reshape(bs, 1, 16)

    pad_n = _NPAD - num_p
    mpT = jnp.pad(jnp.transpose(model_points, (0, 2, 1)),
                  ((0, 0), (0, 0), (0, pad_n)))
    tgtT = jnp.pad(jnp.transpose(target, (0, 2, 1)),
                   ((0, 0), (0, 0), (0, pad_n)), constant_values=_PADVAL)
    tgt_p = jnp.pad(target, ((0, 0), (0, pad_n), (0, 0)),
                    constant_values=_PADVAL)

    out = pl.pallas_call(
        _loss_kernel,
        grid=(bs,),
        in_specs=[
            pl.BlockSpec((1, 1, 16), lambda b: (b, 0, 0), memory_space=pltpu.SMEM),
            pl.BlockSpec((1, 3, _NPAD), lambda b: (b, 0, 0)),
            pl.BlockSpec((1, 3, _NPAD), lambda b: (b, 0, 0)),
            pl.BlockSpec((1, _NPAD, 3), lambda b: (b, 0, 0)),
        ],
        out_specs=pl.BlockSpec((1, 1, 1), lambda b: (b, 0, 0)),
        out_shape=jax.ShapeDtypeStruct((bs, 1, 1), jnp.float32),
        scratch_shapes=[pltpu.VMEM((_NPAD, 1), jnp.float32)],
    )(params, mpT, tgtT, tgt_p)

    return out[:, 0, 0] / jnp.float32(num_p)


# speedup vs baseline: 1.2433x; 1.2433x over previous
import jax
import jax.numpy as jnp
from jax.experimental import pallas as pl
from jax.experimental.pallas import tpu as pltpu

_BS = 64
_N = 3000
_NPAD = 3072
_LC = 1024
_N_LC = _NPAD // _LC
_JC = 8
_SYM = (12, 15, 18, 19, 20)
_PADVAL = 1e15


def _loss_kernel(params_ref, mpT_ref, tgtT_ref, tgt_ref, out_ref, r2_ref):
    mpx = mpT_ref[0, 0:1, :]
    mpy = mpT_ref[0, 1:2, :]
    mpz = mpT_ref[0, 2:3, :]

    def p(k):
        return params_ref[0, 0, k]

    tfx = mpx * p(0) + mpy * p(3) + mpz * p(6) + p(9)
    tfy = mpx * p(1) + mpy * p(4) + mpz * p(7) + p(10)
    tfz = mpx * p(2) + mpy * p(5) + mpz * p(8) + p(11)

    lane = jax.lax.broadcasted_iota(jnp.int32, (1, _NPAD), 1)
    lvalid = (lane < _N).astype(jnp.float32)

    m = p(12)

    @pl.when(m > 0.5)
    def _sym():
        tg0 = tgt_ref[0, :, 0:1]
        tg1 = tgt_ref[0, :, 1:2]
        tg2 = tgt_ref[0, :, 2:3]
        r2_ref[:, :] = tg0 * tg0 + tg1 * tg1 + tg2 * tg2

        total = jnp.zeros((1, 1), dtype=jnp.float32)
        for c in range(_N_LC):
            sl = slice(c * _LC, (c + 1) * _LC)
            bx = jnp.broadcast_to(-2.0 * tfx[:, sl], (_JC, _LC))
            by = jnp.broadcast_to(-2.0 * tfy[:, sl], (_JC, _LC))
            bz = jnp.broadcast_to(-2.0 * tfz[:, sl], (_JC, _LC))

            def body(j, acc):
                r0 = j * _JC
                cx = tgt_ref[0, pl.ds(r0, _JC), 0:1]
                cy = tgt_ref[0, pl.ds(r0, _JC), 1:2]
                cz = tgt_ref[0, pl.ds(r0, _JC), 2:3]
                cr = r2_ref[pl.ds(r0, _JC), 0:1]
                v = cx * bx + cy * by
                v = v + cz * bz
                v = v + cr
                return jnp.minimum(acc, v)

            minacc = jax.lax.fori_loop(
                0, _NPAD // _JC, body,
                jnp.full((_JC, _LC), jnp.inf, dtype=jnp.float32),
                unroll=8)
            mrow = jnp.min(minacc, axis=0, keepdims=True)
            q2 = (tfx[:, sl] * tfx[:, sl] + tfy[:, sl] * tfy[:, sl]
                  + tfz[:, sl] * tfz[:, sl])
            d2 = jnp.maximum(mrow + q2, 0.0)
            total = total + jnp.sum(jnp.sqrt(d2) * lvalid[:, sl],
                                    axis=1, keepdims=True)
        out_ref[0] = total

    @pl.when(m <= 0.5)
    def _plain():
        dx = tfx - tgtT_ref[0, 0:1, :]
        dy = tfy - tgtT_ref[0, 1:2, :]
        dz = tfz - tgtT_ref[0, 2:3, :]
        d2 = dx * dx + dy * dy + dz * dz
        s = jnp.sum(jnp.sqrt(d2) * lvalid, axis=1, keepdims=True)
        out_ref[0] = s


def kernel(pred_r, pred_t, target, model_points, idx):
    bs, num_p, _ = target.shape

    q = pred_r / jnp.linalg.norm(pred_r, axis=1, keepdims=True)
    w, x, y, z = q[:, 0], q[:, 1], q[:, 2], q[:, 3]
    r00 = 1.0 - 2.0 * (y * y + z * z)
    r01 = 2.0 * (x * y - w * z)
    r02 = 2.0 * (x * z + w * y)
    r10 = 2.0 * (x * y + w * z)
    r11 = 1.0 - 2.0 * (x * x + z * z)
    r12 = 2.0 * (y * z - w * x)
    r20 = 2.0 * (x * z - w * y)
    r21 = 2.0 * (y * z + w * x)
    r22 = 1.0 - 2.0 * (x * x + y * y)
    sym = jnp.asarray(_SYM, dtype=idx.dtype)
    mask = (idx[:, 0][:, None] == sym[None, :]).any(axis=1).astype(jnp.float32)
    zeros = jnp.zeros_like(w)
    params = jnp.stack(
        [r00, r01, r02, r10, r11, r12, r20, r21, r22,
         pred_t[:, 0], pred_t[:, 1], pred_t[:, 2], mask, zeros, zeros, zeros],
        axis=1).reshape(bs, 1, 16)

    pad_n = _NPAD - num_p
    mpT = jnp.pad(jnp.transpose(model_points, (0, 2, 1)),
                  ((0, 0), (0, 0), (0, pad_n)))
    tgtT = jnp.pad(jnp.transpose(target, (0, 2, 1)),
                   ((0, 0), (0, 0), (0, pad_n)), constant_values=_PADVAL)
    tgt_p = jnp.pad(target, ((0, 0), (0, pad_n), (0, 0)),
                    constant_values=_PADVAL)

    out = pl.pallas_call(
        _loss_kernel,
        grid=(bs,),
        in_specs=[
            pl.BlockSpec((1, 1, 16), lambda b: (b, 0, 0), memory_space=pltpu.SMEM),
            pl.BlockSpec((1, 3, _NPAD), lambda b: (b, 0, 0)),
            pl.BlockSpec((1, 3, _NPAD), lambda b: (b, 0, 0)),
            pl.BlockSpec((1, _NPAD, 3), lambda b: (b, 0, 0)),
        ],
        out_specs=pl.BlockSpec((1, 1, 1), lambda b: (b, 0, 0)),
        out_shape=jax.ShapeDtypeStruct((bs, 1, 1), jnp.float32),
        scratch_shapes=[pltpu.VMEM((_NPAD, 1), jnp.float32)],
    )(params, mpT, tgtT, tgt_p)

    return out[:, 0, 0] / jnp.float32(num_p)
